# baseline (speedup 1.0000x reference)
import functools

import jax
import jax.numpy as jnp
from jax import lax
from jax.experimental import pallas as pl
from jax.experimental.pallas import tpu as pltpu
from jax.experimental.pallas import tpu_sc as plsc

MIN_RATING = 1.0
MAX_RATING = 5.0

B = 16384
D = 32
NC = 2
NS = 16
NW = NC * NS
BPW = B // NW
CHUNK = 128
NCH = BPW // CHUNK
L = 16
NG = BPW // L

_mesh = plsc.VectorSubcoreMesh(core_axis_name="c", subcore_axis_name="s")


@functools.partial(
    pl.kernel,
    out_type=jax.ShapeDtypeStruct((B,), jnp.float32),
    mesh=_mesh,
    scratch_types=[
        pltpu.VMEM((NCH, CHUNK), jnp.int32),
        pltpu.VMEM((NCH, CHUNK), jnp.int32),
        pltpu.VMEM((BPW, D), jnp.float32),
        pltpu.VMEM((BPW, D), jnp.float32),
        pltpu.VMEM((BPW,), jnp.float32),
        pltpu.VMEM((BPW,), jnp.float32),
        pltpu.VMEM((BPW,), jnp.float32),
        pltpu.SemaphoreType.DMA,
    ],
    compiler_params=pltpu.CompilerParams(
        needs_layout_passes=False, use_tc_tiling_on_sc=False),
)
def _lmf_sc(uidx_hbm, midx_hbm, uw_hbm, ub_hbm, mw_hbm, mb_hbm, out_hbm,
            uidx_v, midx_v, uw_v, mw_v, ub_v, mb_v, out_v, sem):
    wid = lax.axis_index("s") * NC + lax.axis_index("c")
    base = wid * BPW

    pltpu.sync_copy(uidx_hbm.at[pl.ds(wid * NCH, NCH)], uidx_v)
    pltpu.sync_copy(midx_hbm.at[pl.ds(wid * NCH, NCH)], midx_v)

    copies = []
    for j in range(NCH):
        sl = pl.ds(j * CHUNK, CHUNK)
        copies.append(pltpu.async_copy(uw_hbm.at[uidx_v.at[j]], uw_v.at[sl], sem))
        copies.append(pltpu.async_copy(mw_hbm.at[midx_v.at[j]], mw_v.at[sl], sem))
        copies.append(pltpu.async_copy(ub_hbm.at[uidx_v.at[j]], ub_v.at[sl], sem))
        copies.append(pltpu.async_copy(mb_hbm.at[midx_v.at[j]], mb_v.at[sl], sem))
    for c in copies:
        c.wait()

    def group(g, carry):
        rows = lax.iota(jnp.int32, L) + g * L
        acc = jnp.zeros((L,), jnp.float32)
        for j in range(D):
            col = jnp.full((L,), j, jnp.int32)
            u = plsc.load_gather(uw_v, [rows, col])
            m = plsc.load_gather(mw_v, [rows, col])
            acc = acc + u * m
        sl = pl.ds(g * L, L)
        x = acc + ub_v[sl] + mb_v[sl]
        y = 1.0 / (1.0 + jnp.exp(-x))
        out_v[sl] = y * (MAX_RATING - MIN_RATING) + MIN_RATING
        return carry

    lax.fori_loop(0, NG, group, None)
    pltpu.sync_copy(out_v, out_hbm.at[pl.ds(base, BPW)])


def kernel(users, movies, user_weights, user_bias, movie_weights, movie_bias):
    uidx = users.reshape(-1).astype(jnp.int32).reshape(NW * NCH, CHUNK)
    midx = movies.reshape(-1).astype(jnp.int32).reshape(NW * NCH, CHUNK)
    out = _lmf_sc(uidx, midx,
                  user_weights, user_bias.reshape(-1),
                  movie_weights, movie_bias.reshape(-1))
    return out.reshape(B, 1)
